# baseline (speedup 1.0000x reference)
import jax
import jax.numpy as jnp
from jax.experimental import pallas as pl
from jax.experimental.pallas import tpu as pltpu

_TIME_TILE = 2048


def _weights_kernel(zcat_ref, trig_ref, a_ref):
    logits = jax.lax.dot_general(
        zcat_ref[...], trig_ref[...],
        dimension_numbers=(((1,), (0,)), ((), ())),
        preferred_element_type=jnp.float32,
        precision=jax.lax.Precision.HIGHEST)
    m = jnp.max(logits, axis=-1, keepdims=True)
    e = jnp.exp(logits - m)
    a_ref[...] = (e * (1.0 / jnp.sum(e, axis=-1, keepdims=True))).astype(
        a_ref.dtype)


def _apply_kernel(a_ref, x_ref, o_ref):
    x = x_ref[...].astype(jnp.bfloat16)
    o_ref[...] = jax.lax.dot_general(
        a_ref[...], x,
        dimension_numbers=(((1,), (0,)), ((), ())),
        preferred_element_type=jnp.float32)


def kernel(z_re, z_im, cos_buf, sin_buf, X):
    D1, K, _ = z_re.shape
    C = cos_buf.shape[-1]
    B, Cx, T = X.shape
    K2 = K * K

    zcat = jnp.concatenate(
        [z_re.reshape(D1, K2), z_im.reshape(D1, K2)], axis=1)
    trig = jnp.concatenate(
        [cos_buf.reshape(K2, C), sin_buf.reshape(K2, C)], axis=0)

    D1p = max(8, -(-D1 // 8) * 8)
    if D1p != D1:
        zcat = jnp.pad(zcat, ((0, D1p - D1), (0, 0)))

    a = pl.pallas_call(
        _weights_kernel,
        out_shape=jax.ShapeDtypeStruct((D1p, C), jnp.bfloat16),
    )(zcat, trig)

    tT = min(_TIME_TILE, T)
    nT = pl.cdiv(T, tT)

    out = pl.pallas_call(
        _apply_kernel,
        out_shape=jax.ShapeDtypeStruct((B, D1p, T), X.dtype),
        grid=(B, nT),
        in_specs=[
            pl.BlockSpec((D1p, C), lambda b, t: (0, 0)),
            pl.BlockSpec((pl.Squeezed(), C, tT), lambda b, t: (b, 0, t)),
        ],
        out_specs=pl.BlockSpec((pl.Squeezed(), D1p, tT),
                               lambda b, t: (b, 0, t)),
        compiler_params=pltpu.CompilerParams(
            dimension_semantics=("parallel", "parallel"),
            vmem_limit_bytes=48 << 20),
    )(a, X)

    if D1p != D1:
        out = out[:, :D1, :]
    return out
